# Initial kernel scaffold; baseline (speedup 1.0000x reference)
import jax
import jax.numpy as jnp
from jax.experimental import pallas as pl


def kernel(x, edge_index, W1q, b1q, W1k, b1k, W1v, b1v, W1s, b1s, W2q, b2q, W2k, b2k, W2v, b2v, W2s, b2s, W3q, b3q, W3k, b3k, W3v, b3v, W3s, b3s, Wc, bc):
    raise NotImplementedError("write your pallas kernel here")


# speedup vs baseline: 1.3132x; 1.3132x over previous
import functools

import jax
import jax.numpy as jnp
from jax import lax
from jax.experimental import pallas as pl
from jax.experimental.pallas import tpu as pltpu

_N = 10000
_E = 320000
_ROWS = 2000


def _proj_body(h_ref, wq_ref, bq_ref, wk_ref, bk_ref, wv_ref, bv_ref,
               ws_ref, bs_ref, q_ref, k_ref, v_ref, s_ref, rq_ref, rk_ref):
    h = h_ref[...]
    hi = lax.Precision.HIGHEST
    q = jnp.dot(h, wq_ref[...], precision=hi) + bq_ref[...]
    k = jnp.dot(h, wk_ref[...], precision=hi) + bk_ref[...]
    v = jnp.dot(h, wv_ref[...], precision=hi) + bv_ref[...]
    s = jnp.dot(h, ws_ref[...], precision=hi) + bs_ref[...]
    q_ref[...] = q
    k_ref[...] = k
    v_ref[...] = v
    s_ref[...] = s
    rq_ref[...] = jnp.sqrt(jnp.sum(q * q, axis=1, keepdims=True))
    rk_ref[...] = jnp.sqrt(jnp.sum(k * k, axis=1, keepdims=True))


def _project(h, wq, bq, wk, bk, wv, bv, ws, bs):
    n, fi = h.shape
    fo = wq.shape[1]
    grid = n // _ROWS
    row_spec = pl.BlockSpec((_ROWS, fi), lambda i: (i, 0))
    out_spec = pl.BlockSpec((_ROWS, fo), lambda i: (i, 0))
    w_spec = pl.BlockSpec((fi, fo), lambda i: (0, 0))
    b_spec = pl.BlockSpec((fo,), lambda i: (0,))
    nrm_spec = pl.BlockSpec((_ROWS, 1), lambda i: (i, 0))
    f32 = jnp.float32
    return pl.pallas_call(
        _proj_body,
        grid=(grid,),
        in_specs=[row_spec, w_spec, b_spec, w_spec, b_spec, w_spec, b_spec,
                  w_spec, b_spec],
        out_specs=[out_spec, out_spec, out_spec, out_spec, nrm_spec, nrm_spec],
        out_shape=[jax.ShapeDtypeStruct((n, fo), f32)] * 4
        + [jax.ShapeDtypeStruct((n, 1), f32)] * 2,
    )(h, wq, bq, wk, bk, wv, bv, ws, bs)


def _finalize_body(agg_ref, den_ref, s_ref, o_ref, *, relu):
    h = agg_ref[...] / (den_ref[...] + 1e-16) + s_ref[...]
    o_ref[...] = jnp.maximum(h, 0.0) if relu else h


def _finalize(agg, den, s, relu):
    n, fo = agg.shape
    grid = n // _ROWS
    spec = pl.BlockSpec((_ROWS, fo), lambda i: (i, 0))
    dspec = pl.BlockSpec((_ROWS, 1), lambda i: (i, 0))
    return pl.pallas_call(
        functools.partial(_finalize_body, relu=relu),
        grid=(grid,),
        in_specs=[spec, dspec, spec],
        out_specs=spec,
        out_shape=jax.ShapeDtypeStruct((n, fo), jnp.float32),
    )(agg, den, s)


def _edge_phase(q, k, v, mhat, src, dst, rc):
    n = q.shape[0]
    logits = jnp.sum(q[dst] * k[src], axis=-1) * rc
    e = jnp.exp(logits - mhat[dst])
    den = jax.ops.segment_sum(e, dst, num_segments=n)
    agg = jax.ops.segment_sum(v[src] * e[:, None], dst, num_segments=n)
    return agg, den


def _classifier_body(h_ref, w_ref, b_ref, o_ref):
    o_ref[...] = (
        jnp.dot(h_ref[...], w_ref[...], precision=lax.Precision.HIGHEST)
        + b_ref[...]
    )


def _classifier(h, w, b):
    n, fi = h.shape
    fo = w.shape[1]
    grid = n // _ROWS
    return pl.pallas_call(
        _classifier_body,
        grid=(grid,),
        in_specs=[pl.BlockSpec((_ROWS, fi), lambda i: (i, 0)),
                  pl.BlockSpec((fi, fo), lambda i: (0, 0)),
                  pl.BlockSpec((fo,), lambda i: (0,))],
        out_specs=pl.BlockSpec((_ROWS, fo), lambda i: (i, 0)),
        out_shape=jax.ShapeDtypeStruct((n, fo), jnp.float32),
    )(h, w, b)


def kernel(x, edge_index, W1q, b1q, W1k, b1k, W1v, b1v, W1s, b1s,
           W2q, b2q, W2k, b2k, W2v, b2v, W2s, b2s,
           W3q, b3q, W3k, b3k, W3v, b3v, W3s, b3s, Wc, bc):
    src = edge_index[0]
    dst = edge_index[1]
    layers = [
        (W1q, b1q, W1k, b1k, W1v, b1v, W1s, b1s, True),
        (W2q, b2q, W2k, b2k, W2v, b2v, W2s, b2s, True),
        (W3q, b3q, W3k, b3k, W3v, b3v, W3s, b3s, False),
    ]
    h = x
    for (wq, bq, wk, bk, wv, bv, ws, bs, relu) in layers:
        c = wq.shape[1]
        rc = 1.0 / jnp.sqrt(jnp.float32(c))
        q, k, v, s, rq, rk = _project(h, wq, bq, wk, bk, wv, bv, ws, bs)
        kmax = jnp.max(rk)
        mhat = (rq[:, 0] * kmax) * rc
        agg, den = _edge_phase(q, k, v, mhat, src, dst, rc)
        h = _finalize(agg, den[:, None], s, relu)
    return _classifier(h, Wc, bc)


# speedup vs baseline: 7.0917x; 5.4005x over previous
import functools

import jax
import jax.numpy as jnp
from jax import lax
from jax.experimental import pallas as pl
from jax.experimental.pallas import tpu as pltpu
from jax.experimental.pallas import tpu_sc as plsc

_N = 10000
_E = 320000
_ROWS = 2000
_B = 128
_NBLK = _E // _B
_W = 32
_BLK_PER_W = _NBLK // _W
_BLK_PER_S = _NBLK // 16
_NROW = 625


def _proj_body(h_ref, wq_ref, bq_ref, wk_ref, bk_ref, wv_ref, bv_ref,
               ws_ref, bs_ref, q_ref, k_ref, v_ref, s_ref, rq_ref, rk_ref):
    h = h_ref[...]
    hi = lax.Precision.HIGHEST
    q = jnp.dot(h, wq_ref[...], precision=hi) + bq_ref[...]
    k = jnp.dot(h, wk_ref[...], precision=hi) + bk_ref[...]
    v = jnp.dot(h, wv_ref[...], precision=hi) + bv_ref[...]
    s = jnp.dot(h, ws_ref[...], precision=hi) + bs_ref[...]
    q_ref[...] = q
    k_ref[...] = k
    if v_ref.shape[0] == 2:
        half = v.shape[1] // 2
        v_ref[0] = v[:, :half]
        v_ref[1] = v[:, half:]
    else:
        v_ref[...] = v
    s_ref[...] = s
    rq_ref[...] = jnp.sqrt(jnp.sum(q * q, axis=1, keepdims=True))
    rk_ref[...] = jnp.sqrt(jnp.sum(k * k, axis=1, keepdims=True))


def _project(h, wq, bq, wk, bk, wv, bv, ws, bs, split_v):
    n, fi = h.shape
    fo = wq.shape[1]
    grid = n // _ROWS
    row_spec = pl.BlockSpec((_ROWS, fi), lambda i: (i, 0))
    out_spec = pl.BlockSpec((_ROWS, fo), lambda i: (i, 0))
    if split_v:
        vs_spec = pl.BlockSpec((2, _ROWS, fo // 2), lambda i: (0, i, 0))
        v_shape = (2, n, fo // 2)
    else:
        vs_spec = out_spec
        v_shape = (n, fo)
    w_spec = pl.BlockSpec((fi, fo), lambda i: (0, 0))
    b_spec = pl.BlockSpec((fo,), lambda i: (0,))
    nrm_spec = pl.BlockSpec((_ROWS, 1), lambda i: (i, 0))
    f32 = jnp.float32
    return pl.pallas_call(
        _proj_body,
        grid=(grid,),
        in_specs=[row_spec, w_spec, b_spec, w_spec, b_spec, w_spec, b_spec,
                  w_spec, b_spec],
        out_specs=[out_spec, out_spec, vs_spec, out_spec, nrm_spec, nrm_spec],
        out_shape=[jax.ShapeDtypeStruct((n, fo), f32),
                   jax.ShapeDtypeStruct((n, fo), f32),
                   jax.ShapeDtypeStruct(v_shape, f32),
                   jax.ShapeDtypeStruct((n, fo), f32),
                   jax.ShapeDtypeStruct((n, 1), f32),
                   jax.ShapeDtypeStruct((n, 1), f32)],
    )(h, wq, bq, wk, bk, wv, bv, ws, bs)


def _den_combine_body(d_ref, o_ref):
    o_ref[...] = (d_ref[0] + d_ref[1])[:, None]


def _den_combine(den):
    n = den.shape[1]
    return pl.pallas_call(
        _den_combine_body,
        out_shape=jax.ShapeDtypeStruct((n, 1), jnp.float32),
    )(den)


def _finalize_body(agg_ref, den_ref, s_ref, o_ref, *, relu, cat):
    if cat:
        agg = jnp.concatenate([agg_ref[0], agg_ref[1]], axis=1)
    else:
        agg = agg_ref[0] + agg_ref[1]
    h = agg / (den_ref[...] + 1e-16) + s_ref[...]
    o_ref[...] = jnp.maximum(h, 0.0) if relu else h


def _finalize(agg, den1, s, relu, cat):
    _, n, w = agg.shape
    fo = 2 * w if cat else w
    grid = n // _ROWS
    spec = pl.BlockSpec((_ROWS, fo), lambda i: (i, 0))
    return pl.pallas_call(
        functools.partial(_finalize_body, relu=relu, cat=cat),
        grid=(grid,),
        in_specs=[pl.BlockSpec((2, _ROWS, w), lambda i: (0, i, 0)),
                  pl.BlockSpec((_ROWS, 1), lambda i: (i, 0)),
                  spec],
        out_specs=spec,
        out_shape=jax.ShapeDtypeStruct((n, fo), jnp.float32),
    )(agg, den1, s)


def _sc_mesh():
    return plsc.VectorSubcoreMesh(core_axis_name="c", subcore_axis_name="s")


_SC_PARAMS = pltpu.CompilerParams(needs_layout_passes=False)


def _edge_ev_sc(q, k, mhat, src, dst, rc):
    n, d = q.shape
    f32 = jnp.float32
    nchunk = d // 16

    @functools.partial(
        pl.kernel,
        out_type=[jax.ShapeDtypeStruct((_E,), f32),
                  jax.ShapeDtypeStruct((2, n), f32)],
        mesh=_sc_mesh(),
        compiler_params=_SC_PARAMS,
        scratch_types=[
            pltpu.VMEM((_B,), jnp.int32),
            pltpu.VMEM((_B,), jnp.int32),
            pltpu.VMEM((_B, d), f32),
            pltpu.VMEM((_B, d), f32),
            pltpu.VMEM((_B,), f32),
            pltpu.VMEM((16, 16), f32),
            pltpu.VMEM((n,), f32),
            pltpu.VMEM((2000,), f32),
            pltpu.VMEM_SHARED((n,), f32),
        ],
    )
    def body(q_hbm, k_hbm, mhat_hbm, src_hbm, dst_hbm, ev_hbm, den_hbm,
             srcb, dstb, qrows, krows, evb, accst, mloc, zbuf, dsh):
        ci = lax.axis_index("c")
        si = lax.axis_index("s")
        w = si * 2 + ci
        iota = lax.iota(jnp.int32, 16)
        zero16 = jnp.zeros((16,), f32)

        @pl.when(si == 0)
        def _():
            @pl.loop(0, 125)
            def _(i):
                zbuf[pl.ds(i * 16, 16)] = zero16
            for t in range(5):
                pltpu.sync_copy(zbuf, dsh.at[pl.ds(t * 2000, 2000)])

        pltpu.sync_copy(mhat_hbm, mloc)
        plsc.subcore_barrier()

        def block(b):
            off = b * _B
            pltpu.sync_copy(src_hbm.at[pl.ds(off, _B)], srcb)
            pltpu.sync_copy(dst_hbm.at[pl.ds(off, _B)], dstb)
            pltpu.sync_copy(q_hbm.at[dstb], qrows)
            pltpu.sync_copy(k_hbm.at[srcb], krows)

            @pl.loop(0, _B // 16)
            def _(g):
                base = g * 16
                for e16 in range(16):
                    e = base + e16
                    acc = qrows[e, pl.ds(0, 16)] * krows[e, pl.ds(0, 16)]
                    for j in range(1, nchunk):
                        acc = acc + (qrows[e, pl.ds(j * 16, 16)]
                                     * krows[e, pl.ds(j * 16, 16)])
                    accst[e16, :] = acc
                lv = zero16
                for j in range(16):
                    lv = lv + plsc.load_gather(
                        accst, [iota, jnp.full((16,), j, jnp.int32)])
                dstv = dstb[pl.ds(base, 16)]
                mh = plsc.load_gather(mloc, [dstv])
                evb[pl.ds(base, 16)] = jnp.exp(lv * f32(rc) - mh)

            pltpu.sync_copy(evb, ev_hbm.at[pl.ds(off, _B)])
            pltpu.sync_copy(evb, dsh.at[dstb], add=True)

        @pl.loop(0, _BLK_PER_W)
        def _(t):
            block(w + t * _W)

        @pl.when(w < _NBLK - _BLK_PER_W * _W)
        def _():
            block(_BLK_PER_W * _W + w)

        plsc.subcore_barrier()

        @pl.when(si == 0)
        def _():
            pltpu.sync_copy(dsh, den_hbm.at[ci])

    return body(q, k, mhat, src, dst)


def _edge_agg_sc(vperm, ev, src, dst, split_features):
    f = vperm.shape[1]
    n = _N
    f32 = jnp.float32
    if split_features:
        nfull, nextra = _NBLK // 16, _NBLK % 16
    else:
        nfull, nextra = _NBLK // 32, (_NBLK // 2) % 16

    @functools.partial(
        pl.kernel,
        out_type=jax.ShapeDtypeStruct((2, n, f), f32),
        mesh=_sc_mesh(),
        compiler_params=_SC_PARAMS,
        scratch_types=[
            pltpu.VMEM((_B,), jnp.int32),
            pltpu.VMEM((_B,), jnp.int32),
            pltpu.VMEM((_B, f), f32),
            pltpu.VMEM((_B,), f32),
            pltpu.VMEM((200, f), f32),
            pltpu.VMEM_SHARED((n, f), f32),
        ],
    )
    def body(v_hbm, ev_hbm, src_hbm, dst_hbm, agg_hbm,
             srcb, dstb, vrows, evb, zrows, ash):
        ci = lax.axis_index("c")
        si = lax.axis_index("s")
        zero16 = jnp.zeros((16,), f32)
        roff = si * 1000

        @pl.when(si < 10)
        def _():
            @pl.loop(0, 200)
            def _(i):
                for j in range(f // 16):
                    zrows[i, pl.ds(j * 16, 16)] = zero16
            for t in range(5):
                pltpu.sync_copy(zrows, ash.at[pl.ds(roff + t * 200, 200)])

        plsc.subcore_barrier()

        def block(b):
            off = b * _B
            pltpu.sync_copy(src_hbm.at[pl.ds(off, _B)], srcb)
            pltpu.sync_copy(dst_hbm.at[pl.ds(off, _B)], dstb)
            pltpu.sync_copy(ev_hbm.at[pl.ds(off, _B)], evb)
            if split_features:
                coff = ci * n
                for j in range(_B // 16):
                    sl = pl.ds(j * 16, 16)
                    srcb[sl] = srcb[sl] + coff
            pltpu.sync_copy(v_hbm.at[srcb], vrows)

            @pl.loop(0, _B // 16)
            def _(g):
                evv = evb[pl.ds(g * 16, 16)]
                for e16 in range(16):
                    e = g * 16 + e16
                    ev_s = evv[e16]
                    for j in range(f // 16):
                        sl = pl.ds(j * 16, 16)
                        vrows[e, sl] = vrows[e, sl] * ev_s

            pltpu.sync_copy(vrows, ash.at[dstb], add=True)

        base = 0 if split_features else ci * (_NBLK // 2)

        @pl.loop(0, nfull)
        def _(t):
            block(base + si + t * 16)

        @pl.when(si < nextra)
        def _():
            block(base + nfull * 16 + si)

        plsc.subcore_barrier()

        @pl.when(si < 10)
        def _():
            for t in range(5):
                sl = pl.ds(roff + t * 200, 200)
                pltpu.sync_copy(ash.at[sl], agg_hbm.at[ci, sl])

    return body(vperm, ev, src, dst)


def _classifier_body(h_ref, w_ref, b_ref, o_ref):
    o_ref[...] = (
        jnp.dot(h_ref[...], w_ref[...], precision=lax.Precision.HIGHEST)
        + b_ref[...]
    )


def _classifier(h, w, b):
    n, fi = h.shape
    fo = w.shape[1]
    grid = n // _ROWS
    return pl.pallas_call(
        _classifier_body,
        grid=(grid,),
        in_specs=[pl.BlockSpec((_ROWS, fi), lambda i: (i, 0)),
                  pl.BlockSpec((fi, fo), lambda i: (0, 0)),
                  pl.BlockSpec((fo,), lambda i: (0,))],
        out_specs=pl.BlockSpec((_ROWS, fo), lambda i: (i, 0)),
        out_shape=jax.ShapeDtypeStruct((n, fo), jnp.float32),
    )(h, w, b)


def kernel(x, edge_index, W1q, b1q, W1k, b1k, W1v, b1v, W1s, b1s,
           W2q, b2q, W2k, b2k, W2v, b2v, W2s, b2s,
           W3q, b3q, W3k, b3k, W3v, b3v, W3s, b3s, Wc, bc):
    src = edge_index[0]
    dst = edge_index[1]
    layers = [
        (W1q, b1q, W1k, b1k, W1v, b1v, W1s, b1s, True),
        (W2q, b2q, W2k, b2k, W2v, b2v, W2s, b2s, True),
        (W3q, b3q, W3k, b3k, W3v, b3v, W3s, b3s, False),
    ]
    h = x
    for (wq, bq, wk, bk, wv, bv, ws, bs, relu) in layers:
        c = wq.shape[1]
        split = c > 128
        rc = 1.0 / float(c) ** 0.5
        q, k, v, s, rq, rk = _project(h, wq, bq, wk, bk, wv, bv, ws, bs,
                                      split_v=split)
        kmax = jnp.max(rk)
        mhat = (rq[:, 0] * kmax) * rc
        ev, den = _edge_ev_sc(q, k, mhat, src, dst, rc)
        vperm = v.reshape(2 * _N, c // 2) if split else v
        agg = _edge_agg_sc(vperm, ev, src, dst, split_features=split)
        h = _finalize(agg, _den_combine(den), s, relu, cat=split)
    return _classifier(h, Wc, bc)


# speedup vs baseline: 12.9368x; 1.8242x over previous
import functools

import jax
import jax.numpy as jnp
from jax import lax
from jax.experimental import pallas as pl
from jax.experimental.pallas import tpu as pltpu
from jax.experimental.pallas import tpu_sc as plsc

_N = 10000
_E = 320000
_ROWS = 2000
_B = 128
_NBLK = _E // _B
_W = 32
_BLK_PER_W = _NBLK // _W
_BLK_PER_S = _NBLK // 16
_NROW = 625


def _proj_body(h_ref, wq_ref, bq_ref, wk_ref, bk_ref, wv_ref, bv_ref,
               ws_ref, bs_ref, q_ref, k_ref, v_ref, s_ref, rq_ref, rk_ref):
    h = h_ref[...]
    hi = lax.Precision.HIGHEST
    q = jnp.dot(h, wq_ref[...], precision=hi) + bq_ref[...]
    k = jnp.dot(h, wk_ref[...], precision=hi) + bk_ref[...]
    v = jnp.dot(h, wv_ref[...], precision=hi) + bv_ref[...]
    s = jnp.dot(h, ws_ref[...], precision=hi) + bs_ref[...]
    q_ref[...] = q
    k_ref[...] = k
    if v_ref.shape[0] == 2:
        half = v.shape[1] // 2
        v_ref[0] = v[:, :half]
        v_ref[1] = v[:, half:]
    else:
        v_ref[...] = v
    s_ref[...] = s
    rq_ref[...] = jnp.sqrt(jnp.sum(q * q, axis=1, keepdims=True))
    rk_ref[...] = jnp.sqrt(jnp.sum(k * k, axis=1, keepdims=True))


def _project(h, wq, bq, wk, bk, wv, bv, ws, bs, split_v):
    n, fi = h.shape
    fo = wq.shape[1]
    grid = n // _ROWS
    row_spec = pl.BlockSpec((_ROWS, fi), lambda i: (i, 0))
    out_spec = pl.BlockSpec((_ROWS, fo), lambda i: (i, 0))
    if split_v:
        vs_spec = pl.BlockSpec((2, _ROWS, fo // 2), lambda i: (0, i, 0))
        v_shape = (2, n, fo // 2)
    else:
        vs_spec = out_spec
        v_shape = (n, fo)
    w_spec = pl.BlockSpec((fi, fo), lambda i: (0, 0))
    b_spec = pl.BlockSpec((fo,), lambda i: (0,))
    nrm_spec = pl.BlockSpec((_ROWS, 1), lambda i: (i, 0))
    f32 = jnp.float32
    return pl.pallas_call(
        _proj_body,
        grid=(grid,),
        in_specs=[row_spec, w_spec, b_spec, w_spec, b_spec, w_spec, b_spec,
                  w_spec, b_spec],
        out_specs=[out_spec, out_spec, vs_spec, out_spec, nrm_spec, nrm_spec],
        out_shape=[jax.ShapeDtypeStruct((n, fo), f32),
                   jax.ShapeDtypeStruct((n, fo), f32),
                   jax.ShapeDtypeStruct(v_shape, f32),
                   jax.ShapeDtypeStruct((n, fo), f32),
                   jax.ShapeDtypeStruct((n, 1), f32),
                   jax.ShapeDtypeStruct((n, 1), f32)],
    )(h, wq, bq, wk, bk, wv, bv, ws, bs)


def _den_combine_body(d_ref, o_ref):
    o_ref[...] = (d_ref[0] + d_ref[1])[:, None]


def _den_combine(den):
    n = den.shape[1]
    return pl.pallas_call(
        _den_combine_body,
        out_shape=jax.ShapeDtypeStruct((n, 1), jnp.float32),
    )(den)


def _finalize_body(agg_ref, den_ref, s_ref, o_ref, *, relu, cat):
    if cat:
        agg = jnp.concatenate([agg_ref[0], agg_ref[1]], axis=1)
    else:
        agg = agg_ref[0] + agg_ref[1]
    h = agg / (den_ref[...] + 1e-16) + s_ref[...]
    o_ref[...] = jnp.maximum(h, 0.0) if relu else h


def _finalize(agg, den1, s, relu, cat):
    _, n, w = agg.shape
    fo = 2 * w if cat else w
    grid = n // _ROWS
    spec = pl.BlockSpec((_ROWS, fo), lambda i: (i, 0))
    return pl.pallas_call(
        functools.partial(_finalize_body, relu=relu, cat=cat),
        grid=(grid,),
        in_specs=[pl.BlockSpec((2, _ROWS, w), lambda i: (0, i, 0)),
                  pl.BlockSpec((_ROWS, 1), lambda i: (i, 0)),
                  spec],
        out_specs=spec,
        out_shape=jax.ShapeDtypeStruct((n, fo), jnp.float32),
    )(agg, den1, s)


def _sc_mesh():
    return plsc.VectorSubcoreMesh(core_axis_name="c", subcore_axis_name="s")


_SC_PARAMS = pltpu.CompilerParams(needs_layout_passes=False)


def _edge_ev_sc(q, k, mhat, src, dst, rc):
    n, d = q.shape
    f32 = jnp.float32
    nchunk = d // 16
    B = 128 if d <= 128 else 64
    per_w = _E // _W
    nfull = per_w // B
    tail = per_w - nfull * B

    @functools.partial(
        pl.kernel,
        out_type=[jax.ShapeDtypeStruct((_E,), f32),
                  jax.ShapeDtypeStruct((2, n), f32)],
        mesh=_sc_mesh(),
        compiler_params=_SC_PARAMS,
        scratch_types=[
            pltpu.VMEM((per_w,), jnp.int32),
            pltpu.VMEM((per_w,), jnp.int32),
            pltpu.VMEM((B,), jnp.int32),
            pltpu.VMEM((tail,), jnp.int32),
            pltpu.VMEM((2, B, d), f32),
            pltpu.VMEM((2, B, d), f32),
            pltpu.VMEM((tail, d), f32),
            pltpu.VMEM((tail, d), f32),
            pltpu.VMEM((B,), f32),
            pltpu.VMEM((16, 16), f32),
            pltpu.VMEM((n,), f32),
            pltpu.VMEM((2000,), f32),
            pltpu.VMEM_SHARED((n,), f32),
            pltpu.SemaphoreType.DMA,
            pltpu.SemaphoreType.DMA,
        ],
    )
    def body(q_hbm, k_hbm, mhat_hbm, src_hbm, dst_hbm, ev_hbm, den_hbm,
             srcall, dstall, dstb, dsttail, qbuf, kbuf, qtail, ktail,
             evb, accst, mloc, zbuf, dsh, sem0, sem1):
        ci = lax.axis_index("c")
        si = lax.axis_index("s")
        w = si * 2 + ci
        base = w * per_w
        iota = lax.iota(jnp.int32, 16)
        zero16 = jnp.zeros((16,), f32)
        sems = (sem0, sem1)

        @pl.when(si == 0)
        def _():
            @pl.loop(0, 125)
            def _(i):
                zbuf[pl.ds(i * 16, 16)] = zero16
            for t in range(5):
                pltpu.sync_copy(zbuf, dsh.at[pl.ds(t * 2000, 2000)])

        pltpu.sync_copy(mhat_hbm, mloc)
        pltpu.sync_copy(src_hbm.at[pl.ds(base, per_w)], srcall)
        pltpu.sync_copy(dst_hbm.at[pl.ds(base, per_w)], dstall)
        plsc.subcore_barrier()

        def gathers(t, bi):
            sl = pl.ds(t * B, B)
            return (pltpu.make_async_copy(q_hbm.at[dstall.at[sl]],
                                          qbuf.at[bi], sems[bi]),
                    pltpu.make_async_copy(k_hbm.at[srcall.at[sl]],
                                          kbuf.at[bi], sems[bi]))

        def start(t, bi):
            a, b = gathers(t, bi)
            a.start()
            b.start()

        def wait(t, bi):
            a, b = gathers(t, bi)
            a.wait()
            b.wait()

        def dot16(qr, kr, ebase):
            for e16 in range(16):
                e = ebase + e16
                acc = qr[e, pl.ds(0, 16)] * kr[e, pl.ds(0, 16)]
                for j in range(1, nchunk):
                    acc = acc + (qr[e, pl.ds(j * 16, 16)]
                                 * kr[e, pl.ds(j * 16, 16)])
                accst[e16, :] = acc
            lv = zero16
            for j in range(16):
                lv = lv + plsc.load_gather(
                    accst, [iota, jnp.full((16,), j, jnp.int32)])
            return lv

        def compute(t, bi):
            qr = qbuf.at[bi]
            kr = kbuf.at[bi]

            @pl.loop(0, B // 16)
            def _(g):
                gbase = g * 16
                lv = dot16(qr, kr, gbase)
                dstv = dstall[pl.ds(t * B + gbase, 16)]
                mh = plsc.load_gather(mloc, [dstv])
                evb[pl.ds(gbase, 16)] = jnp.exp(lv * f32(rc) - mh)

            for j in range(B // 16):
                dstb[pl.ds(j * 16, 16)] = dstall[pl.ds(t * B + j * 16, 16)]
            pltpu.sync_copy(evb, ev_hbm.at[pl.ds(base + t * B, B)])
            pltpu.sync_copy(evb, dsh.at[dstb], add=True)

        start(0, 0)

        @pl.loop(0, nfull // 2)
        def _(i):
            t0 = i * 2
            start(t0 + 1, 1)
            wait(t0, 0)
            compute(t0, 0)

            @pl.when(i < nfull // 2 - 1)
            def _():
                start(t0 + 2, 0)

            wait(t0 + 1, 1)
            compute(t0 + 1, 1)

        toff = nfull * B
        tsl = pl.ds(toff, tail)
        pltpu.sync_copy(q_hbm.at[dstall.at[tsl]], qtail)
        pltpu.sync_copy(k_hbm.at[srcall.at[tsl]], ktail)
        lv = dot16(qtail, ktail, 0)
        dstv = dstall[tsl]
        mh = plsc.load_gather(mloc, [dstv])
        evb[pl.ds(0, tail)] = jnp.exp(lv * f32(rc) - mh)
        for j in range(tail // 16):
            dsttail[pl.ds(j * 16, 16)] = dstall[pl.ds(toff + j * 16, 16)]
        pltpu.sync_copy(evb.at[pl.ds(0, tail)], ev_hbm.at[pl.ds(base + toff, tail)])
        pltpu.sync_copy(evb.at[pl.ds(0, tail)], dsh.at[dsttail], add=True)

        plsc.subcore_barrier()

        @pl.when(si == 0)
        def _():
            pltpu.sync_copy(dsh, den_hbm.at[ci])

    return body(q, k, mhat, src, dst)


def _edge_agg_sc(vperm, ev, src, dst, split_features):
    f = vperm.shape[1]
    n = _N
    f32 = jnp.float32
    B = 128
    per_t = _E // 16 if split_features else _E // 32
    nfull = per_t // B
    tail = per_t - nfull * B

    @functools.partial(
        pl.kernel,
        out_type=jax.ShapeDtypeStruct((2, n, f), f32),
        mesh=_sc_mesh(),
        compiler_params=_SC_PARAMS,
        scratch_types=[
            pltpu.VMEM((2, B), jnp.int32),
            pltpu.VMEM((2, B), jnp.int32),
            pltpu.VMEM((2, B), f32),
            pltpu.VMEM((tail,), jnp.int32),
            pltpu.VMEM((tail,), jnp.int32),
            pltpu.VMEM((tail,), f32),
            pltpu.VMEM((2, B, f), f32),
            pltpu.VMEM((40, f), f32),
            pltpu.VMEM_SHARED((n, f), f32),
            pltpu.SemaphoreType.DMA,
            pltpu.SemaphoreType.DMA,
            pltpu.SemaphoreType.DMA,
            pltpu.SemaphoreType.DMA,
        ],
    )
    def body(v_hbm, ev_hbm, src_hbm, dst_hbm, agg_hbm,
             srcb, dstb, evb, srctail, dsttail, evtail, vbuf, zrows, ash,
             sem0, sem1, isem0, isem1):
        ci = lax.axis_index("c")
        si = lax.axis_index("s")
        zero16 = jnp.zeros((16,), f32)
        sems = (sem0, sem1)
        isems = (isem0, isem1)
        roff = si * 1000

        @pl.when(si < 10)
        def _():
            @pl.loop(0, 40)
            def _(i):
                for j in range(f // 16):
                    zrows[i, pl.ds(j * 16, 16)] = zero16
            for t in range(25):
                pltpu.sync_copy(zrows, ash.at[pl.ds(roff + t * 40, 40)])

        base = (si if split_features else si * 2 + ci) * per_t
        plsc.subcore_barrier()

        def idx_dmas(t, bi):
            sl = pl.ds(base + t * B, B)
            return (pltpu.make_async_copy(src_hbm.at[sl], srcb.at[bi],
                                          isems[bi]),
                    pltpu.make_async_copy(dst_hbm.at[sl], dstb.at[bi],
                                          isems[bi]),
                    pltpu.make_async_copy(ev_hbm.at[sl], evb.at[bi],
                                          isems[bi]))

        def idx_start(t, bi):
            for cp in idx_dmas(t, bi):
                cp.start()

        def idx_wait(t, bi):
            for cp in idx_dmas(t, bi):
                cp.wait()
            if split_features:
                coff = ci * n
                for j in range(B // 16):
                    sl = pl.ds(j * 16, 16)
                    srcb.at[bi][sl] = srcb.at[bi][sl] + coff

        def gather(t, bi):
            return pltpu.make_async_copy(v_hbm.at[srcb.at[bi]],
                                         vbuf.at[bi], sems[bi])

        def compute(t, bi):
            vb = vbuf.at[bi]
            eb = evb.at[bi]

            @pl.loop(0, B // 16)
            def _(g):
                evv = eb[pl.ds(g * 16, 16)]
                for e16 in range(16):
                    e = g * 16 + e16
                    ev_s = evv[e16]
                    for j in range(f // 16):
                        sl = pl.ds(j * 16, 16)
                        vb[e, sl] = vb[e, sl] * ev_s

            pltpu.sync_copy(vb, ash.at[dstb.at[bi]], add=True)

        idx_start(0, 0)
        idx_wait(0, 0)
        gather(0, 0).start()
        idx_start(1, 1)

        @pl.loop(0, nfull // 2)
        def _(i):
            t0 = i * 2
            last = i >= nfull // 2 - 1

            idx_wait(t0 + 1, 1)
            gather(t0 + 1, 1).start()
            gather(t0, 0).wait()
            compute(t0, 0)

            @pl.when(jnp.logical_not(last))
            def _():
                idx_start(t0 + 2, 0)

            gather(t0 + 1, 1).wait()
            compute(t0 + 1, 1)

            @pl.when(jnp.logical_not(last))
            def _():
                idx_wait(t0 + 2, 0)
                gather(t0 + 2, 0).start()
                idx_start(t0 + 3, 1)

        toff = nfull * B
        tsl = pl.ds(base + toff, tail)
        vb0 = vbuf.at[0]
        pltpu.sync_copy(src_hbm.at[tsl], srctail)
        pltpu.sync_copy(dst_hbm.at[tsl], dsttail)
        pltpu.sync_copy(ev_hbm.at[tsl], evtail)
        if split_features:
            for j in range(tail // 16):
                sl = pl.ds(j * 16, 16)
                srctail[sl] = srctail[sl] + ci * n
        pltpu.sync_copy(v_hbm.at[srctail], vb0.at[pl.ds(0, tail)])

        for g in range(tail // 16):
            evv = evtail[pl.ds(g * 16, 16)]
            for e16 in range(16):
                e = g * 16 + e16
                ev_s = evv[e16]
                for j in range(f // 16):
                    sl = pl.ds(j * 16, 16)
                    vb0[e, sl] = vb0[e, sl] * ev_s

        pltpu.sync_copy(vb0.at[pl.ds(0, tail)], ash.at[dsttail], add=True)

        plsc.subcore_barrier()

        @pl.when(si < 10)
        def _():
            for t in range(5):
                sl = pl.ds(roff + t * 200, 200)
                pltpu.sync_copy(ash.at[sl], agg_hbm.at[ci, sl])

    return body(vperm, ev, src, dst)


def _classifier_body(h_ref, w_ref, b_ref, o_ref):
    o_ref[...] = (
        jnp.dot(h_ref[...], w_ref[...], precision=lax.Precision.HIGHEST)
        + b_ref[...]
    )


def _classifier(h, w, b):
    n, fi = h.shape
    fo = w.shape[1]
    grid = n // _ROWS
    return pl.pallas_call(
        _classifier_body,
        grid=(grid,),
        in_specs=[pl.BlockSpec((_ROWS, fi), lambda i: (i, 0)),
                  pl.BlockSpec((fi, fo), lambda i: (0, 0)),
                  pl.BlockSpec((fo,), lambda i: (0,))],
        out_specs=pl.BlockSpec((_ROWS, fo), lambda i: (i, 0)),
        out_shape=jax.ShapeDtypeStruct((n, fo), jnp.float32),
    )(h, w, b)


def kernel(x, edge_index, W1q, b1q, W1k, b1k, W1v, b1v, W1s, b1s,
           W2q, b2q, W2k, b2k, W2v, b2v, W2s, b2s,
           W3q, b3q, W3k, b3k, W3v, b3v, W3s, b3s, Wc, bc):
    src = edge_index[0]
    dst = edge_index[1]
    layers = [
        (W1q, b1q, W1k, b1k, W1v, b1v, W1s, b1s, True),
        (W2q, b2q, W2k, b2k, W2v, b2v, W2s, b2s, True),
        (W3q, b3q, W3k, b3k, W3v, b3v, W3s, b3s, False),
    ]
    h = x
    for (wq, bq, wk, bk, wv, bv, ws, bs, relu) in layers:
        c = wq.shape[1]
        split = c > 128
        rc = 1.0 / float(c) ** 0.5
        q, k, v, s, rq, rk = _project(h, wq, bq, wk, bk, wv, bv, ws, bs,
                                      split_v=split)
        kmax = jnp.max(rk)
        mhat = (rq[:, 0] * kmax) * rc
        ev, den = _edge_ev_sc(q, k, mhat, src, dst, rc)
        vperm = v.reshape(2 * _N, c // 2) if split else v
        agg = _edge_agg_sc(vperm, ev, src, dst, split_features=split)
        h = _finalize(agg, _den_combine(den), s, relu, cat=split)
    return _classifier(h, Wc, bc)


# speedup vs baseline: 13.5270x; 1.0456x over previous
import functools

import jax
import jax.numpy as jnp
from jax import lax
from jax.experimental import pallas as pl
from jax.experimental.pallas import tpu as pltpu
from jax.experimental.pallas import tpu_sc as plsc

_N = 10000
_E = 320000
_ROWS = 2000
_B = 128
_NBLK = _E // _B
_W = 32
_BLK_PER_W = _NBLK // _W
_BLK_PER_S = _NBLK // 16
_NROW = 625


def _proj_body(h_ref, wq_ref, bq_ref, wk_ref, bk_ref, wv_ref, bv_ref,
               ws_ref, bs_ref, q_ref, k_ref, v_ref, s_ref, rq_ref, rk_ref):
    h = h_ref[...]
    hi = lax.Precision.HIGHEST
    q = jnp.dot(h, wq_ref[...], precision=hi) + bq_ref[...]
    k = jnp.dot(h, wk_ref[...], precision=hi) + bk_ref[...]
    v = jnp.dot(h, wv_ref[...], precision=hi) + bv_ref[...]
    s = jnp.dot(h, ws_ref[...], precision=hi) + bs_ref[...]
    if q_ref.dtype == jnp.int32:
        def _pack(a):
            abf = a.astype(jnp.bfloat16)
            h = abf.shape[1] // 2
            lo = lax.bitcast_convert_type(abf[:, :h], jnp.uint16)
            hi = lax.bitcast_convert_type(abf[:, h:], jnp.uint16)
            packed = lo.astype(jnp.uint32) | (hi.astype(jnp.uint32) << 16)
            return (lax.bitcast_convert_type(packed, jnp.int32),
                    abf.astype(jnp.float32))

        qp, q = _pack(q)
        kp, k = _pack(k)
        q_ref[...] = qp
        k_ref[...] = kp
    else:
        q_ref[...] = q
        k_ref[...] = k
    if v_ref.shape[0] == 2:
        half = v.shape[1] // 2
        v_ref[0] = v[:, :half]
        v_ref[1] = v[:, half:]
    else:
        v_ref[...] = v
    s_ref[...] = s
    rq_ref[...] = jnp.sqrt(jnp.sum(q * q, axis=1, keepdims=True))
    rk_ref[...] = jnp.sqrt(jnp.sum(k * k, axis=1, keepdims=True))


def _project(h, wq, bq, wk, bk, wv, bv, ws, bs, split_v):
    n, fi = h.shape
    fo = wq.shape[1]
    grid = n // _ROWS
    row_spec = pl.BlockSpec((_ROWS, fi), lambda i: (i, 0))
    out_spec = pl.BlockSpec((_ROWS, fo), lambda i: (i, 0))
    pack_qk = fo > 128
    qk_spec = pl.BlockSpec((_ROWS, fo // 2), lambda i: (i, 0)) if pack_qk \
        else out_spec
    qk_type = (jax.ShapeDtypeStruct((n, fo // 2), jnp.int32) if pack_qk
               else jax.ShapeDtypeStruct((n, fo), jnp.float32))
    if split_v:
        vs_spec = pl.BlockSpec((2, _ROWS, fo // 2), lambda i: (0, i, 0))
        v_shape = (2, n, fo // 2)
    else:
        vs_spec = out_spec
        v_shape = (n, fo)
    w_spec = pl.BlockSpec((fi, fo), lambda i: (0, 0))
    b_spec = pl.BlockSpec((fo,), lambda i: (0,))
    nrm_spec = pl.BlockSpec((_ROWS, 1), lambda i: (i, 0))
    f32 = jnp.float32
    return pl.pallas_call(
        _proj_body,
        grid=(grid,),
        in_specs=[row_spec, w_spec, b_spec, w_spec, b_spec, w_spec, b_spec,
                  w_spec, b_spec],
        out_specs=[qk_spec, qk_spec, vs_spec, out_spec, nrm_spec, nrm_spec],
        out_shape=[qk_type,
                   qk_type,
                   jax.ShapeDtypeStruct(v_shape, f32),
                   jax.ShapeDtypeStruct((n, fo), f32),
                   jax.ShapeDtypeStruct((n, 1), f32),
                   jax.ShapeDtypeStruct((n, 1), f32)],
    )(h, wq, bq, wk, bk, wv, bv, ws, bs)


def _den_combine_body(d_ref, o_ref):
    o_ref[...] = (d_ref[0] + d_ref[1])[:, None]


def _den_combine(den):
    n = den.shape[1]
    return pl.pallas_call(
        _den_combine_body,
        out_shape=jax.ShapeDtypeStruct((n, 1), jnp.float32),
    )(den)


def _finalize_body(agg_ref, den_ref, s_ref, o_ref, *, relu, cat):
    if cat:
        agg = jnp.concatenate([agg_ref[0], agg_ref[1]], axis=1)
    else:
        agg = agg_ref[0] + agg_ref[1]
    h = agg / (den_ref[...] + 1e-16) + s_ref[...]
    o_ref[...] = jnp.maximum(h, 0.0) if relu else h


def _finalize(agg, den1, s, relu, cat):
    _, n, w = agg.shape
    fo = 2 * w if cat else w
    grid = n // _ROWS
    spec = pl.BlockSpec((_ROWS, fo), lambda i: (i, 0))
    return pl.pallas_call(
        functools.partial(_finalize_body, relu=relu, cat=cat),
        grid=(grid,),
        in_specs=[pl.BlockSpec((2, _ROWS, w), lambda i: (0, i, 0)),
                  pl.BlockSpec((_ROWS, 1), lambda i: (i, 0)),
                  spec],
        out_specs=spec,
        out_shape=jax.ShapeDtypeStruct((n, fo), jnp.float32),
    )(agg, den1, s)


def _sc_mesh():
    return plsc.VectorSubcoreMesh(core_axis_name="c", subcore_axis_name="s")


_SC_PARAMS = pltpu.CompilerParams(needs_layout_passes=False)


def _edge_ev_sc(q, k, mhat, src, dst, rc):
    n, d = q.shape
    f32 = jnp.float32
    bf16 = jnp.bfloat16
    packed = q.dtype == jnp.int32
    nchunk = d // 16
    B = 128
    per_w = _E // _W
    nfull = per_w // B
    tail = per_w - nfull * B
    qk_dtype = jnp.int32 if packed else f32

    @functools.partial(
        pl.kernel,
        out_type=[jax.ShapeDtypeStruct((_E,), f32),
                  jax.ShapeDtypeStruct((2, n), f32)],
        mesh=_sc_mesh(),
        compiler_params=_SC_PARAMS,
        scratch_types=[
            pltpu.VMEM((per_w,), jnp.int32),
            pltpu.VMEM((per_w,), jnp.int32),
            pltpu.VMEM((B,), jnp.int32),
            pltpu.VMEM((tail,), jnp.int32),
            pltpu.VMEM((2, B, d), qk_dtype),
            pltpu.VMEM((2, B, d), qk_dtype),
            pltpu.VMEM((tail, d), qk_dtype),
            pltpu.VMEM((tail, d), qk_dtype),
            pltpu.VMEM((B,), f32),
            pltpu.VMEM((16, 16), f32),
            pltpu.VMEM((n,), f32),
            pltpu.VMEM((2000,), f32),
            pltpu.VMEM_SHARED((n,), f32),
            pltpu.SemaphoreType.DMA,
            pltpu.SemaphoreType.DMA,
        ],
    )
    def body(q_hbm, k_hbm, mhat_hbm, src_hbm, dst_hbm, ev_hbm, den_hbm,
             srcall, dstall, dstb, dsttail, qbuf, kbuf, qtail, ktail,
             evb, accst, mloc, zbuf, dsh, sem0, sem1):
        ci = lax.axis_index("c")
        si = lax.axis_index("s")
        w = si * 2 + ci
        base = w * per_w
        iota = lax.iota(jnp.int32, 16)
        zero16 = jnp.zeros((16,), f32)
        sems = (sem0, sem1)

        @pl.when(si == 0)
        def _():
            @pl.loop(0, 125)
            def _(i):
                zbuf[pl.ds(i * 16, 16)] = zero16
            for t in range(5):
                pltpu.sync_copy(zbuf, dsh.at[pl.ds(t * 2000, 2000)])

        pltpu.sync_copy(mhat_hbm, mloc)
        pltpu.sync_copy(src_hbm.at[pl.ds(base, per_w)], srcall)
        pltpu.sync_copy(dst_hbm.at[pl.ds(base, per_w)], dstall)
        plsc.subcore_barrier()

        def gathers(t, bi):
            sl = pl.ds(t * B, B)
            return (pltpu.make_async_copy(q_hbm.at[dstall.at[sl]],
                                          qbuf.at[bi], sems[bi]),
                    pltpu.make_async_copy(k_hbm.at[srcall.at[sl]],
                                          kbuf.at[bi], sems[bi]))

        def start(t, bi):
            a, b = gathers(t, bi)
            a.start()
            b.start()

        def wait(t, bi):
            a, b = gathers(t, bi)
            a.wait()
            b.wait()

        def dot16(qr, kr, ebase):
            for e16 in range(16):
                e = ebase + e16
                acc = None
                for j in range(nchunk):
                    sl = pl.ds(j * 16, 16)
                    if packed:
                        qa, qb = plsc.unpack(
                            plsc.bitcast(qr[e, sl], bf16),
                            format=plsc.PackFormat.INTERLEAVED,
                            preferred_element_type=f32)
                        ka, kb = plsc.unpack(
                            plsc.bitcast(kr[e, sl], bf16),
                            format=plsc.PackFormat.INTERLEAVED,
                            preferred_element_type=f32)
                        part = qa * ka + qb * kb
                    else:
                        part = qr[e, sl] * kr[e, sl]
                    acc = part if acc is None else acc + part
                accst[e16, :] = acc
            lv = zero16
            for j in range(16):
                lv = lv + plsc.load_gather(
                    accst, [iota, jnp.full((16,), j, jnp.int32)])
            return lv

        def compute(t, bi):
            qr = qbuf.at[bi]
            kr = kbuf.at[bi]

            @pl.loop(0, B // 16)
            def _(g):
                gbase = g * 16
                lv = dot16(qr, kr, gbase)
                dstv = dstall[pl.ds(t * B + gbase, 16)]
                mh = plsc.load_gather(mloc, [dstv])
                evb[pl.ds(gbase, 16)] = jnp.exp(lv * f32(rc) - mh)

            for j in range(B // 16):
                dstb[pl.ds(j * 16, 16)] = dstall[pl.ds(t * B + j * 16, 16)]
            pltpu.sync_copy(evb, ev_hbm.at[pl.ds(base + t * B, B)])
            pltpu.sync_copy(evb, dsh.at[dstb], add=True)

        start(0, 0)

        @pl.loop(0, nfull // 2)
        def _(i):
            t0 = i * 2
            start(t0 + 1, 1)
            wait(t0, 0)
            compute(t0, 0)

            @pl.when(i < nfull // 2 - 1)
            def _():
                start(t0 + 2, 0)

            wait(t0 + 1, 1)
            compute(t0 + 1, 1)

        toff = nfull * B
        tsl = pl.ds(toff, tail)
        pltpu.sync_copy(q_hbm.at[dstall.at[tsl]], qtail)
        pltpu.sync_copy(k_hbm.at[srcall.at[tsl]], ktail)
        lv = dot16(qtail, ktail, 0)
        dstv = dstall[tsl]
        mh = plsc.load_gather(mloc, [dstv])
        evb[pl.ds(0, tail)] = jnp.exp(lv * f32(rc) - mh)
        for j in range(tail // 16):
            dsttail[pl.ds(j * 16, 16)] = dstall[pl.ds(toff + j * 16, 16)]
        pltpu.sync_copy(evb.at[pl.ds(0, tail)], ev_hbm.at[pl.ds(base + toff, tail)])
        pltpu.sync_copy(evb.at[pl.ds(0, tail)], dsh.at[dsttail], add=True)

        plsc.subcore_barrier()

        @pl.when(si == 0)
        def _():
            pltpu.sync_copy(dsh, den_hbm.at[ci])

    return body(q, k, mhat, src, dst)


def _edge_agg_sc(vperm, ev, src, dst, split_features):
    f = vperm.shape[1]
    n = _N
    f32 = jnp.float32
    B = 128
    per_t = _E // 16 if split_features else _E // 32
    nfull = per_t // B
    tail = per_t - nfull * B

    @functools.partial(
        pl.kernel,
        out_type=jax.ShapeDtypeStruct((2, n, f), f32),
        mesh=_sc_mesh(),
        compiler_params=_SC_PARAMS,
        scratch_types=[
            pltpu.VMEM((2, B), jnp.int32),
            pltpu.VMEM((2, B), jnp.int32),
            pltpu.VMEM((2, B), f32),
            pltpu.VMEM((tail,), jnp.int32),
            pltpu.VMEM((tail,), jnp.int32),
            pltpu.VMEM((tail,), f32),
            pltpu.VMEM((2, B, f), f32),
            pltpu.VMEM((40, f), f32),
            pltpu.VMEM_SHARED((n, f), f32),
            pltpu.SemaphoreType.DMA,
            pltpu.SemaphoreType.DMA,
            pltpu.SemaphoreType.DMA,
            pltpu.SemaphoreType.DMA,
        ],
    )
    def body(v_hbm, ev_hbm, src_hbm, dst_hbm, agg_hbm,
             srcb, dstb, evb, srctail, dsttail, evtail, vbuf, zrows, ash,
             sem0, sem1, isem0, isem1):
        ci = lax.axis_index("c")
        si = lax.axis_index("s")
        zero16 = jnp.zeros((16,), f32)
        sems = (sem0, sem1)
        isems = (isem0, isem1)
        roff = si * 1000

        @pl.when(si < 10)
        def _():
            @pl.loop(0, 40)
            def _(i):
                for j in range(f // 16):
                    zrows[i, pl.ds(j * 16, 16)] = zero16
            for t in range(25):
                pltpu.sync_copy(zrows, ash.at[pl.ds(roff + t * 40, 40)])

        base = (si if split_features else si * 2 + ci) * per_t
        plsc.subcore_barrier()

        def idx_dmas(t, bi):
            sl = pl.ds(base + t * B, B)
            return (pltpu.make_async_copy(src_hbm.at[sl], srcb.at[bi],
                                          isems[bi]),
                    pltpu.make_async_copy(dst_hbm.at[sl], dstb.at[bi],
                                          isems[bi]),
                    pltpu.make_async_copy(ev_hbm.at[sl], evb.at[bi],
                                          isems[bi]))

        def idx_start(t, bi):
            for cp in idx_dmas(t, bi):
                cp.start()

        def idx_wait(t, bi):
            for cp in idx_dmas(t, bi):
                cp.wait()
            if split_features:
                coff = ci * n
                for j in range(B // 16):
                    sl = pl.ds(j * 16, 16)
                    srcb.at[bi][sl] = srcb.at[bi][sl] + coff

        def gather(t, bi):
            return pltpu.make_async_copy(v_hbm.at[srcb.at[bi]],
                                         vbuf.at[bi], sems[bi])

        def compute(t, bi):
            vb = vbuf.at[bi]
            eb = evb.at[bi]

            @pl.loop(0, B // 16)
            def _(g):
                evv = eb[pl.ds(g * 16, 16)]
                for e16 in range(16):
                    e = g * 16 + e16
                    ev_s = evv[e16]
                    for j in range(f // 16):
                        sl = pl.ds(j * 16, 16)
                        vb[e, sl] = vb[e, sl] * ev_s

            pltpu.sync_copy(vb, ash.at[dstb.at[bi]], add=True)

        idx_start(0, 0)
        idx_wait(0, 0)
        gather(0, 0).start()
        idx_start(1, 1)

        @pl.loop(0, nfull // 2)
        def _(i):
            t0 = i * 2
            last = i >= nfull // 2 - 1

            idx_wait(t0 + 1, 1)
            gather(t0 + 1, 1).start()
            gather(t0, 0).wait()
            compute(t0, 0)

            @pl.when(jnp.logical_not(last))
            def _():
                idx_start(t0 + 2, 0)

            gather(t0 + 1, 1).wait()
            compute(t0 + 1, 1)

            @pl.when(jnp.logical_not(last))
            def _():
                idx_wait(t0 + 2, 0)
                gather(t0 + 2, 0).start()
                idx_start(t0 + 3, 1)

        toff = nfull * B
        tsl = pl.ds(base + toff, tail)
        vb0 = vbuf.at[0]
        pltpu.sync_copy(src_hbm.at[tsl], srctail)
        pltpu.sync_copy(dst_hbm.at[tsl], dsttail)
        pltpu.sync_copy(ev_hbm.at[tsl], evtail)
        if split_features:
            for j in range(tail // 16):
                sl = pl.ds(j * 16, 16)
                srctail[sl] = srctail[sl] + ci * n
        pltpu.sync_copy(v_hbm.at[srctail], vb0.at[pl.ds(0, tail)])

        for g in range(tail // 16):
            evv = evtail[pl.ds(g * 16, 16)]
            for e16 in range(16):
                e = g * 16 + e16
                ev_s = evv[e16]
                for j in range(f // 16):
                    sl = pl.ds(j * 16, 16)
                    vb0[e, sl] = vb0[e, sl] * ev_s

        pltpu.sync_copy(vb0.at[pl.ds(0, tail)], ash.at[dsttail], add=True)

        plsc.subcore_barrier()

        @pl.when(si < 10)
        def _():
            for t in range(5):
                sl = pl.ds(roff + t * 200, 200)
                pltpu.sync_copy(ash.at[sl], agg_hbm.at[ci, sl])

    return body(vperm, ev, src, dst)


def _classifier_body(h_ref, w_ref, b_ref, o_ref):
    o_ref[...] = (
        jnp.dot(h_ref[...], w_ref[...], precision=lax.Precision.HIGHEST)
        + b_ref[...]
    )


def _classifier(h, w, b):
    n, fi = h.shape
    fo = w.shape[1]
    grid = n // _ROWS
    return pl.pallas_call(
        _classifier_body,
        grid=(grid,),
        in_specs=[pl.BlockSpec((_ROWS, fi), lambda i: (i, 0)),
                  pl.BlockSpec((fi, fo), lambda i: (0, 0)),
                  pl.BlockSpec((fo,), lambda i: (0,))],
        out_specs=pl.BlockSpec((_ROWS, fo), lambda i: (i, 0)),
        out_shape=jax.ShapeDtypeStruct((n, fo), jnp.float32),
    )(h, w, b)


def kernel(x, edge_index, W1q, b1q, W1k, b1k, W1v, b1v, W1s, b1s,
           W2q, b2q, W2k, b2k, W2v, b2v, W2s, b2s,
           W3q, b3q, W3k, b3k, W3v, b3v, W3s, b3s, Wc, bc):
    src = edge_index[0]
    dst = edge_index[1]
    layers = [
        (W1q, b1q, W1k, b1k, W1v, b1v, W1s, b1s, True),
        (W2q, b2q, W2k, b2k, W2v, b2v, W2s, b2s, True),
        (W3q, b3q, W3k, b3k, W3v, b3v, W3s, b3s, False),
    ]
    h = x
    for (wq, bq, wk, bk, wv, bv, ws, bs, relu) in layers:
        c = wq.shape[1]
        split = c > 128
        rc = 1.0 / float(c) ** 0.5
        q, k, v, s, rq, rk = _project(h, wq, bq, wk, bk, wv, bv, ws, bs,
                                      split_v=split)
        kmax = jnp.max(rk)
        mhat = (rq[:, 0] * kmax) * rc
        ev, den = _edge_ev_sc(q, k, mhat, src, dst, rc)
        vperm = v.reshape(2 * _N, c // 2) if split else v
        agg = _edge_agg_sc(vperm, ev, src, dst, split_features=split)
        h = _finalize(agg, _den_combine(den), s, relu, cat=split)
    return _classifier(h, Wc, bc)
